# baseline (speedup 1.0000x reference)
import jax
import jax.numpy as jnp
from jax.experimental import pallas as pl
from jax.experimental.pallas import tpu as pltpu

_B, _N, _C = 8, 2048, 40
_K = 8


def _loss_kernel(tgt_ref, a_ref, b_ref, pred_ref, out_ref):
    bidx = pl.program_id(0)
    a = a_ref[0]
    bb = b_ref[0]
    an = jnp.sum(a * a, axis=1, keepdims=True)
    bn = jnp.sum(bb * bb, axis=0, keepdims=True)
    cross = jax.lax.dot(a, bb, preferred_element_type=jnp.float32)
    g = jnp.maximum(an + bn - 2.0 * cross, 0.0)
    s1 = jnp.sum(jnp.min(g, axis=1))
    s2 = jnp.sum(jnp.min(g, axis=0))
    t = tgt_ref[bidx]
    pr = pred_ref[0]
    col = jax.lax.broadcasted_iota(jnp.int32, (1, _C), 1)
    pv = jnp.sum(jnp.where(col == t, pr, 0.0))
    val = (s1 + s2) / (_N * _B) - pv / _B
    out_ref[...] = val.reshape(1, 1, 1)


def kernel(reg, point1, pred, target):
    a3 = jnp.pad(reg, ((0, 0), (0, 0), (0, _K - 3)))
    b3 = jnp.pad(point1, ((0, 0), (0, 0), (0, _K - 3))).transpose(0, 2, 1)
    pred3 = pred.reshape(_B, 1, _C)

    grid_spec = pltpu.PrefetchScalarGridSpec(
        num_scalar_prefetch=1,
        grid=(_B,),
        in_specs=[
            pl.BlockSpec((1, _N, _K), lambda b, tgt: (b, 0, 0)),
            pl.BlockSpec((1, _K, _N), lambda b, tgt: (b, 0, 0)),
            pl.BlockSpec((1, 1, _C), lambda b, tgt: (b, 0, 0)),
        ],
        out_specs=pl.BlockSpec((1, 1, 1), lambda b, tgt: (b, 0, 0)),
    )
    out = pl.pallas_call(
        _loss_kernel,
        grid_spec=grid_spec,
        out_shape=jax.ShapeDtypeStruct((_B, 1, 1), jnp.float32),
        compiler_params=pltpu.CompilerParams(
            dimension_semantics=("arbitrary",),
        ),
    )(target, a3, b3, pred3)
    return jnp.sum(out)


# speedup vs baseline: 1.2276x; 1.2276x over previous
import jax
import jax.numpy as jnp
from jax.experimental import pallas as pl
from jax.experimental.pallas import tpu as pltpu

_B, _N, _C = 8, 2048, 40
_K = 8


def _loss_kernel(tgt_ref, a_ref, b_ref, pred_ref, out_ref):
    bidx = pl.program_id(0)
    a = a_ref[0]
    bb = b_ref[0]
    an = jnp.sum(a * a, axis=1, keepdims=True)
    bn = jnp.sum(bb * bb, axis=0, keepdims=True)
    col = jax.lax.broadcasted_iota(jnp.int32, (_N, _K), 1)
    row = jax.lax.broadcasted_iota(jnp.int32, (_K, _N), 0)
    lhs = -2.0 * a + jnp.where(col == 3, an, 0.0) + jnp.where(col == 4, 1.0, 0.0)
    rhs = bb + jnp.where(row == 3, 1.0, 0.0) + jnp.where(row == 4, bn, 0.0)
    g = jax.lax.dot(lhs, rhs, preferred_element_type=jnp.float32)
    m1 = jnp.min(g, axis=1, keepdims=True)
    m2 = jnp.min(g, axis=0, keepdims=True)
    s1 = jnp.sum(jnp.maximum(m1, 0.0))
    s2 = jnp.sum(jnp.maximum(m2, 0.0))
    t = tgt_ref[bidx]
    pr = pred_ref[0]
    pcol = jax.lax.broadcasted_iota(jnp.int32, (1, _C), 1)
    pv = jnp.sum(jnp.where(pcol == t, pr, 0.0))
    val = (s1 + s2) / (_N * _B) - pv / _B
    out_ref[...] = val.reshape(1, 1, 1)


def kernel(reg, point1, pred, target):
    a3 = jnp.pad(reg, ((0, 0), (0, 0), (0, _K - 3)))
    b3 = jnp.pad(point1, ((0, 0), (0, 0), (0, _K - 3))).transpose(0, 2, 1)
    pred3 = pred.reshape(_B, 1, _C)

    grid_spec = pltpu.PrefetchScalarGridSpec(
        num_scalar_prefetch=1,
        grid=(_B,),
        in_specs=[
            pl.BlockSpec((1, _N, _K), lambda b, tgt: (b, 0, 0)),
            pl.BlockSpec((1, _K, _N), lambda b, tgt: (b, 0, 0)),
            pl.BlockSpec((1, 1, _C), lambda b, tgt: (b, 0, 0)),
        ],
        out_specs=pl.BlockSpec((1, 1, 1), lambda b, tgt: (b, 0, 0)),
    )
    out = pl.pallas_call(
        _loss_kernel,
        grid_spec=grid_spec,
        out_shape=jax.ShapeDtypeStruct((_B, 1, 1), jnp.float32),
        compiler_params=pltpu.CompilerParams(
            dimension_semantics=("parallel",),
        ),
    )(target, a3, b3, pred3)
    return jnp.sum(out)


# speedup vs baseline: 1.2861x; 1.0476x over previous
import jax
import jax.numpy as jnp
from jax.experimental import pallas as pl
from jax.experimental.pallas import tpu as pltpu

_B, _N, _C = 8, 2048, 40
_K = 8


def _loss_kernel(tgt_ref, a_ref, b_ref, pred_ref, out_ref):
    col = jax.lax.broadcasted_iota(jnp.int32, (_N, _K), 1)
    row = jax.lax.broadcasted_iota(jnp.int32, (_K, _N), 0)
    pcol = jax.lax.broadcasted_iota(jnp.int32, (1, _C), 1)
    total = jnp.float32(0.0)
    for b in range(_B):
        a = a_ref[b]
        bb = b_ref[b]
        an = jnp.sum(a * a, axis=1, keepdims=True)
        bn = jnp.sum(bb * bb, axis=0, keepdims=True)
        lhs = -2.0 * a + jnp.where(col == 3, an, 0.0) + jnp.where(col == 4, 1.0, 0.0)
        rhs = bb + jnp.where(row == 3, 1.0, 0.0) + jnp.where(row == 4, bn, 0.0)
        g = jax.lax.dot(lhs, rhs, preferred_element_type=jnp.float32)
        m1 = jnp.min(g, axis=1, keepdims=True)
        m2 = jnp.min(g, axis=0, keepdims=True)
        s1 = jnp.sum(jnp.maximum(m1, 0.0))
        s2 = jnp.sum(jnp.maximum(m2, 0.0))
        pv = jnp.sum(jnp.where(pcol == tgt_ref[b], pred_ref[b], 0.0))
        total += (s1 + s2) / (_N * _B) - pv / _B
    out_ref[...] = total.reshape(1, 1)


def kernel(reg, point1, pred, target):
    a3 = jnp.pad(reg, ((0, 0), (0, 0), (0, _K - 3)))
    b3 = jnp.pad(point1, ((0, 0), (0, 0), (0, _K - 3))).transpose(0, 2, 1)
    pred3 = pred.reshape(_B, 1, _C)

    grid_spec = pltpu.PrefetchScalarGridSpec(
        num_scalar_prefetch=1,
        grid=(1,),
        in_specs=[
            pl.BlockSpec((_B, _N, _K), lambda i, tgt: (0, 0, 0)),
            pl.BlockSpec((_B, _K, _N), lambda i, tgt: (0, 0, 0)),
            pl.BlockSpec((_B, 1, _C), lambda i, tgt: (0, 0, 0)),
        ],
        out_specs=pl.BlockSpec((1, 1), lambda i, tgt: (0, 0)),
    )
    out = pl.pallas_call(
        _loss_kernel,
        grid_spec=grid_spec,
        out_shape=jax.ShapeDtypeStruct((1, 1), jnp.float32),
        compiler_params=pltpu.CompilerParams(
            dimension_semantics=("arbitrary",),
        ),
    )(target, a3, b3, pred3)
    return out[0, 0]
